# baseline (speedup 1.0000x reference)
import functools

import jax
import jax.numpy as jnp
import numpy as np
from jax import lax
from jax.experimental import pallas as pl
from jax.experimental.pallas import tpu as pltpu
from jax.experimental.pallas import tpu_sc as plsc

_F = 26
_V = 100000
_D = 16
_B = 4096
_NPAIR = (_F * (_F - 1)) // 2
_NW = 32
_IDXW = 128
_NITEM = _NPAIR * (_B // _IDXW)
_BASE = _NITEM // _NW
_VW = 128


def _sc_ffm(idxa, idxb, table):
    mesh = plsc.VectorSubcoreMesh(core_axis_name="c", subcore_axis_name="s")

    @functools.partial(
        pl.kernel,
        mesh=mesh,
        out_type=jax.ShapeDtypeStruct((_NPAIR, _D, _B), jnp.float32),
        scratch_types=[
            pltpu.VMEM((2, _IDXW), jnp.int32),
            pltpu.VMEM((2, _IDXW), jnp.int32),
            pltpu.VMEM((2, _IDXW), jnp.int32),
            pltpu.VMEM((2, _IDXW), jnp.int32),
            pltpu.VMEM((2 * _IDXW, _VW), jnp.float32),
            pltpu.VMEM((2 * _IDXW, _VW), jnp.float32),
            pltpu.VMEM((2 * _D, _IDXW), jnp.float32),
            pltpu.SemaphoreType.DMA,
            pltpu.SemaphoreType.DMA,
            pltpu.SemaphoreType.DMA,
            pltpu.SemaphoreType.DMA,
            pltpu.SemaphoreType.DMA,
        ],
        compiler_params=pltpu.CompilerParams(
            use_tc_tiling_on_sc=True, needs_layout_passes=False),
    )
    def k(idxa_hbm, idxb_hbm, table_hbm, out_hbm,
          idxa_v, idxb_v, va_v, vb_v, ra_v, rb_v, out_v,
          semi, semg0, semg1, semo0, semo1):
        wid = lax.axis_index("s") * 2 + lax.axis_index("c")
        didx = lax.iota(jnp.int32, _D)
        semg = (semg0, semg1)
        semo = (semo0, semo1)

        def it_of(ci):
            return wid + ci * _NW

        def idx_copy(ci, slot, fire):
            off = it_of(ci) * _IDXW
            for src, dst in ((idxa_hbm, idxa_v), (idxb_hbm, idxb_v)):
                cp = pltpu.make_async_copy(
                    src.at[pl.ds(off, _IDXW)], dst.at[slot], semi)
                if fire:
                    cp.start()
                else:
                    cp.wait()

        def make_view_idx(slot):
            for j in range(_IDXW // _D):
                s = pl.ds(j * _D, _D)
                va_v.at[slot][s] = lax.shift_right_logical(
                    idxa_v.at[slot][s], 3)
                vb_v.at[slot][s] = lax.shift_right_logical(
                    idxb_v.at[slot][s], 3)

        def gathers(slot, fire):
            d = pl.ds(slot * _IDXW, _IDXW)
            for iv, rv in ((va_v, ra_v), (vb_v, rb_v)):
                cp = pltpu.make_async_copy(
                    table_hbm.at[iv.at[slot]], rv.at[d], semg[slot])
                if fire:
                    cp.start()
                else:
                    cp.wait()

        def out_fire(ci, slot):
            it = it_of(ci)
            p = it // (_B // _IDXW)
            sub = it % (_B // _IDXW)
            pltpu.make_async_copy(
                out_v.at[pl.ds(slot * _D, _D)],
                out_hbm.at[p, :, pl.ds(sub * _IDXW, _IDXW)],
                semo[slot]).start()

        def out_drain(slot):
            pltpu.make_async_copy(
                out_v.at[pl.ds(slot * _D, _D)],
                out_hbm.at[0, :, pl.ds(0, _IDXW)],
                semo[slot]).wait()

        def compute(slot):
            rowv = didx + slot * _D
            rbase = slot * _IDXW

            def prod2(q, c):
                l = q * 2
                for k_ in range(2):
                    m = l + k_
                    mv = didx * 0 + m
                    suba = plsc.load_gather(idxa_v, [mv * 0 + slot, mv])
                    subb = plsc.load_gather(idxb_v, [mv * 0 + slot, mv])
                    cola = (suba & 7) * _D + didx
                    colb = (subb & 7) * _D + didx
                    a = plsc.load_gather(ra_v, [mv + rbase, cola])
                    b = plsc.load_gather(rb_v, [mv + rbase, colb])
                    plsc.store_scatter(out_v, [rowv, mv], a * b)
                return c

            lax.fori_loop(0, _IDXW // 2, prod2, 0)

        idx_copy(0, 0, True)
        idx_copy(1, 1, True)
        idx_copy(0, 0, False)
        make_view_idx(0)
        gathers(0, True)

        def body(ci, carry):
            for s_ in (0, 1):
                @pl.when(ci % 2 == s_)
                def _(s_=s_):
                    cur, oth = s_, 1 - s_

                    @pl.when(ci + 1 < _BASE)
                    def _():
                        idx_copy(ci + 1, oth, False)
                        make_view_idx(oth)
                        gathers(oth, True)

                    gathers(cur, False)
                    @pl.when(ci >= 2)
                    def _():
                        out_drain(cur)

                    compute(cur)

                    @pl.when(ci + 2 < _BASE)
                    def _():
                        idx_copy(ci + 2, cur, True)

                    out_fire(ci, cur)
            return carry

        lax.fori_loop(0, _BASE, body, 0)

        out_drain(0)
        out_drain(1)

    return k(idxa, idxb, table)


def kernel(x, tables, offsets):
    xi_t = (x + offsets[None, :]).T
    iu, ju = np.triu_indices(_F, k=1)
    iu = jnp.asarray(iu, jnp.int32)
    ju = jnp.asarray(ju, jnp.int32)
    idxa = (xi_t[iu] + (ju * _V)[:, None]).reshape(_NPAIR * _B)
    idxb = (xi_t[ju] + (iu * _V)[:, None]).reshape(_NPAIR * _B)
    table = tables.reshape(_F * _V * _D // _VW, _VW)
    out = _sc_ffm(idxa, idxb, table)
    return jnp.transpose(out, (2, 0, 1))


# speedup vs baseline: 1.2341x; 1.2341x over previous
import functools

import jax
import jax.numpy as jnp
import numpy as np
from jax import lax
from jax.experimental import pallas as pl
from jax.experimental.pallas import tpu as pltpu
from jax.experimental.pallas import tpu_sc as plsc

_F = 26
_V = 100000
_D = 16
_B = 4096
_NPAIR = (_F * (_F - 1)) // 2
_NW = 32
_IDXW = 128
_G = 8
_CW = _G * _IDXW
_CPP = _B // _CW
_NCHUNK = _NPAIR * _CPP
_BASE = _NCHUNK // _NW
_EXTRA = _NCHUNK % _NW


def _sc_ffm(idxa, idxb, table):
    mesh = plsc.VectorSubcoreMesh(core_axis_name="c", subcore_axis_name="s")

    @functools.partial(
        pl.kernel,
        mesh=mesh,
        out_type=jax.ShapeDtypeStruct((_NPAIR, _D, _B), jnp.float32),
        scratch_types=[
            pltpu.VMEM((2, _CW), jnp.int32),
            pltpu.VMEM((2, _CW), jnp.int32),
            pltpu.VMEM((2 * _CW, _D), jnp.float32),
            pltpu.VMEM((2 * _CW, _D), jnp.float32),
            pltpu.VMEM((2 * _D, _CW), jnp.float32),
            pltpu.SemaphoreType.DMA,
            pltpu.SemaphoreType.DMA,
            pltpu.SemaphoreType.DMA,
            pltpu.SemaphoreType.DMA,
            pltpu.SemaphoreType.DMA,
        ],
        compiler_params=pltpu.CompilerParams(
            use_tc_tiling_on_sc=False, needs_layout_passes=False),
    )
    def k(idxa_hbm, idxb_hbm, table_hbm, out_hbm,
          idxa_v, idxb_v, ra_v, rb_v, out_v, semi, semg0, semg1, semo0, semo1):
        wid = lax.axis_index("s") * 2 + lax.axis_index("c")
        nchunk = _BASE + jnp.where(wid < _EXTRA, 1, 0)
        didx = lax.iota(jnp.int32, _D)
        semg = (semg0, semg1)
        semo = (semo0, semo1)

        def cc_of(ci):
            return wid + ci * _NW

        def idx_copy(ci, slot, fire):
            off = cc_of(ci) * _CW
            for src, dst in ((idxa_hbm, idxa_v), (idxb_hbm, idxb_v)):
                cp = pltpu.make_async_copy(
                    src.at[pl.ds(off, _CW)], dst.at[slot], semi)
                if fire:
                    cp.start()
                else:
                    cp.wait()

        def gathers(slot, fire):
            for g in range(_G):
                s = pl.ds(g * _IDXW, _IDXW)
                d = pl.ds(slot * _CW + g * _IDXW, _IDXW)
                for iv, rv in ((idxa_v, ra_v), (idxb_v, rb_v)):
                    cp = pltpu.make_async_copy(
                        table_hbm.at[iv.at[slot].at[s]], rv.at[d], semg[slot])
                    if fire:
                        cp.start()
                    else:
                        cp.wait()

        def out_fire(ci, slot):
            cc = cc_of(ci)
            p = cc // _CPP
            sub = cc % _CPP
            pltpu.make_async_copy(
                out_v.at[pl.ds(slot * _D, _D)],
                out_hbm.at[p, :, pl.ds(sub * _CW, _CW)],
                semo[slot]).start()

        def out_drain(slot):
            pltpu.make_async_copy(
                out_v.at[pl.ds(slot * _D, _D)],
                out_hbm.at[0, :, pl.ds(0, _CW)],
                semo[slot]).wait()

        def compute(slot):
            rowv = didx + slot * _D
            rbase = slot * _CW

            def prod4(q, c):
                l = q * 4
                for k_ in range(4):
                    mv = didx * 0 + (l + k_)
                    a = plsc.load_gather(ra_v, [mv + rbase, didx])
                    b = plsc.load_gather(rb_v, [mv + rbase, didx])
                    plsc.store_scatter(out_v, [rowv, mv], a * b)
                return c

            lax.fori_loop(0, _CW // 4, prod4, 0)

        idx_copy(0, 0, True)
        idx_copy(1, 1, True)
        idx_copy(0, 0, False)
        gathers(0, True)

        def body(ci, carry):
            for s_ in (0, 1):
                @pl.when(ci % 2 == s_)
                def _(s_=s_):
                    cur, oth = s_, 1 - s_

                    @pl.when(ci + 1 < nchunk)
                    def _():
                        idx_copy(ci + 1, oth, False)
                        gathers(oth, True)

                    gathers(cur, False)
                    @pl.when(ci + 2 < nchunk)
                    def _():
                        idx_copy(ci + 2, cur, True)

                    @pl.when(ci >= 2)
                    def _():
                        out_drain(cur)

                    compute(cur)
                    out_fire(ci, cur)
            return carry

        lax.fori_loop(0, nchunk, body, 0)

        out_drain(0)
        out_drain(1)

    return k(idxa, idxb, table)


def kernel(x, tables, offsets):
    xi_t = (x + offsets[None, :]).T
    iu, ju = np.triu_indices(_F, k=1)
    iu = jnp.asarray(iu, jnp.int32)
    ju = jnp.asarray(ju, jnp.int32)
    idxa = (xi_t[iu] + (ju * _V)[:, None]).reshape(_NPAIR * _B)
    idxb = (xi_t[ju] + (iu * _V)[:, None]).reshape(_NPAIR * _B)
    table = tables.reshape(_F * _V, _D)
    out = _sc_ffm(idxa, idxb, table)
    return jnp.transpose(out, (2, 0, 1))


# speedup vs baseline: 1.4415x; 1.1681x over previous
import functools

import jax
import jax.numpy as jnp
import numpy as np
from jax import lax
from jax.experimental import pallas as pl
from jax.experimental.pallas import tpu as pltpu
from jax.experimental.pallas import tpu_sc as plsc

_F = 26
_V = 100000
_D = 16
_B = 4096
_NPAIR = (_F * (_F - 1)) // 2
_NW = 32
_IDXW = 128
_G = 8
_CW = _G * _IDXW
_CPP = _B // _CW
_NCHUNK = _NPAIR * _CPP
_BASE = _NCHUNK // _NW
_EXTRA = _NCHUNK % _NW


def _sc_ffm(idxa, idxb, table):
    mesh = plsc.VectorSubcoreMesh(core_axis_name="c", subcore_axis_name="s")

    @functools.partial(
        pl.kernel,
        mesh=mesh,
        out_type=jax.ShapeDtypeStruct((_NPAIR, _D, _B), jnp.float32),
        scratch_types=[
            pltpu.VMEM((2, _CW), jnp.int32),
            pltpu.VMEM((2, _CW), jnp.int32),
            pltpu.VMEM((2 * _CW, _D), jnp.float32),
            pltpu.VMEM((2 * _CW, _D), jnp.float32),
            pltpu.VMEM((2 * _D, _CW + 1), jnp.float32),
            pltpu.SemaphoreType.DMA,
            pltpu.SemaphoreType.DMA,
            pltpu.SemaphoreType.DMA,
            pltpu.SemaphoreType.DMA,
            pltpu.SemaphoreType.DMA,
        ],
        compiler_params=pltpu.CompilerParams(
            use_tc_tiling_on_sc=False, needs_layout_passes=False),
    )
    def k(idxa_hbm, idxb_hbm, table_hbm, out_hbm,
          idxa_v, idxb_v, ra_v, rb_v, out_v, semi, semg0, semg1, semo0, semo1):
        wid = lax.axis_index("s") * 2 + lax.axis_index("c")
        nchunk = _BASE + jnp.where(wid < _EXTRA, 1, 0)
        didx = lax.iota(jnp.int32, _D)
        semg = (semg0, semg1)
        semo = (semo0, semo1)

        def cc_of(ci):
            return wid + ci * _NW

        def idx_copy(ci, slot, fire):
            off = cc_of(ci) * _CW
            for src, dst in ((idxa_hbm, idxa_v), (idxb_hbm, idxb_v)):
                cp = pltpu.make_async_copy(
                    src.at[pl.ds(off, _CW)], dst.at[slot], semi)
                if fire:
                    cp.start()
                else:
                    cp.wait()

        def gathers(slot, fire):
            for g in range(_G):
                s = pl.ds(g * _IDXW, _IDXW)
                d = pl.ds(slot * _CW + g * _IDXW, _IDXW)
                for iv, rv in ((idxa_v, ra_v), (idxb_v, rb_v)):
                    cp = pltpu.make_async_copy(
                        table_hbm.at[iv.at[slot].at[s]], rv.at[d], semg[slot])
                    if fire:
                        cp.start()
                    else:
                        cp.wait()

        def out_fire(ci, slot):
            cc = cc_of(ci)
            p = cc // _CPP
            sub = cc % _CPP
            pltpu.make_async_copy(
                out_v.at[pl.ds(slot * _D, _D), pl.ds(0, _CW)],
                out_hbm.at[p, :, pl.ds(sub * _CW, _CW)],
                semo[slot]).start()

        def out_drain(slot):
            pltpu.make_async_copy(
                out_v.at[pl.ds(slot * _D, _D), pl.ds(0, _CW)],
                out_hbm.at[0, :, pl.ds(0, _CW)],
                semo[slot]).wait()

        def compute(slot):
            rowv = didx + slot * _D
            rbase = slot * _CW

            def prod4(q, c):
                l = q * 4
                for k_ in range(4):
                    mv = didx * 0 + (l + k_)
                    a = plsc.load_gather(ra_v, [mv + rbase, didx])
                    b = plsc.load_gather(rb_v, [mv + rbase, didx])
                    plsc.store_scatter(out_v, [rowv, mv], a * b)
                return c

            lax.fori_loop(0, _CW // 4, prod4, 0)

        idx_copy(0, 0, True)
        idx_copy(1, 1, True)
        idx_copy(0, 0, False)
        gathers(0, True)

        def body(ci, carry):
            for s_ in (0, 1):
                @pl.when(ci % 2 == s_)
                def _(s_=s_):
                    cur, oth = s_, 1 - s_

                    @pl.when(ci + 1 < nchunk)
                    def _():
                        idx_copy(ci + 1, oth, False)
                        gathers(oth, True)

                    gathers(cur, False)
                    @pl.when(ci + 2 < nchunk)
                    def _():
                        idx_copy(ci + 2, cur, True)

                    @pl.when(ci >= 2)
                    def _():
                        out_drain(cur)

                    compute(cur)
                    out_fire(ci, cur)
            return carry

        lax.fori_loop(0, nchunk, body, 0)

        out_drain(0)
        out_drain(1)

    return k(idxa, idxb, table)


def kernel(x, tables, offsets):
    xi_t = (x + offsets[None, :]).T
    iu, ju = np.triu_indices(_F, k=1)
    iu = jnp.asarray(iu, jnp.int32)
    ju = jnp.asarray(ju, jnp.int32)
    idxa = (xi_t[iu] + (ju * _V)[:, None]).reshape(_NPAIR * _B)
    idxb = (xi_t[ju] + (iu * _V)[:, None]).reshape(_NPAIR * _B)
    table = tables.reshape(_F * _V, _D)
    out = _sc_ffm(idxa, idxb, table)
    return jnp.transpose(out, (2, 0, 1))


# speedup vs baseline: 1.4450x; 1.0024x over previous
import functools

import jax
import jax.numpy as jnp
import numpy as np
from jax import lax
from jax.experimental import pallas as pl
from jax.experimental.pallas import tpu as pltpu
from jax.experimental.pallas import tpu_sc as plsc

_F = 26
_V = 100000
_D = 16
_B = 4096
_NPAIR = (_F * (_F - 1)) // 2
_NW = 32
_IDXW = 128
_G = 8
_CW = _G * _IDXW
_CPP = _B // _CW
_NCHUNK = _NPAIR * _CPP
_BASE = _NCHUNK // _NW
_EXTRA = _NCHUNK % _NW


def _sc_ffm(idxa, idxb, table):
    mesh = plsc.VectorSubcoreMesh(core_axis_name="c", subcore_axis_name="s")

    @functools.partial(
        pl.kernel,
        mesh=mesh,
        out_type=jax.ShapeDtypeStruct((_NPAIR, _D, _B), jnp.float32),
        scratch_types=[
            pltpu.VMEM((2, _CW), jnp.int32),
            pltpu.VMEM((2, _CW), jnp.int32),
            pltpu.VMEM((2 * _CW, _D), jnp.float32),
            pltpu.VMEM((2 * _CW, _D), jnp.float32),
            pltpu.VMEM((2 * _D, _CW + 1), jnp.float32),
            pltpu.SemaphoreType.DMA,
            pltpu.SemaphoreType.DMA,
            pltpu.SemaphoreType.DMA,
            pltpu.SemaphoreType.DMA,
            pltpu.SemaphoreType.DMA,
        ],
        compiler_params=pltpu.CompilerParams(
            use_tc_tiling_on_sc=False, needs_layout_passes=False),
    )
    def k(idxa_hbm, idxb_hbm, table_hbm, out_hbm,
          idxa_v, idxb_v, ra_v, rb_v, out_v, semi, semg0, semg1, semo0, semo1):
        wid = lax.axis_index("s") * 2 + lax.axis_index("c")
        nchunk = _BASE + jnp.where(wid < _EXTRA, 1, 0)
        didx = lax.iota(jnp.int32, _D)
        semg = (semg0, semg1)
        semo = (semo0, semo1)

        def cc_of(ci):
            return wid + ci * _NW

        def idx_copy(ci, slot, fire):
            off = cc_of(ci) * _CW
            for src, dst in ((idxa_hbm, idxa_v), (idxb_hbm, idxb_v)):
                cp = pltpu.make_async_copy(
                    src.at[pl.ds(off, _CW)], dst.at[slot], semi)
                if fire:
                    cp.start()
                else:
                    cp.wait()

        def gathers(slot, fire):
            for g in range(_G):
                s = pl.ds(g * _IDXW, _IDXW)
                d = pl.ds(slot * _CW + g * _IDXW, _IDXW)
                for iv, rv in ((idxa_v, ra_v), (idxb_v, rb_v)):
                    cp = pltpu.make_async_copy(
                        table_hbm.at[iv.at[slot].at[s]], rv.at[d], semg[slot])
                    if fire:
                        cp.start()
                    else:
                        cp.wait()

        def out_fire(ci, slot):
            cc = cc_of(ci)
            p = cc // _CPP
            sub = cc % _CPP
            pltpu.make_async_copy(
                out_v.at[pl.ds(slot * _D, _D), pl.ds(0, _CW)],
                out_hbm.at[p, :, pl.ds(sub * _CW, _CW)],
                semo[slot]).start()

        def out_drain(slot):
            pltpu.make_async_copy(
                out_v.at[pl.ds(slot * _D, _D), pl.ds(0, _CW)],
                out_hbm.at[0, :, pl.ds(0, _CW)],
                semo[slot]).wait()

        def compute(slot):
            rowv = didx + slot * _D
            rbase = slot * _CW

            def prod8(q, c):
                l = q * 8
                for k_ in range(8):
                    mv = didx * 0 + (l + k_)
                    a = plsc.load_gather(ra_v, [mv + rbase, didx])
                    b = plsc.load_gather(rb_v, [mv + rbase, didx])
                    plsc.store_scatter(out_v, [rowv, mv], a * b)
                return c

            lax.fori_loop(0, _CW // 8, prod8, 0)

        idx_copy(0, 0, True)
        idx_copy(1, 1, True)
        idx_copy(0, 0, False)
        gathers(0, True)

        def body(ci, carry):
            for s_ in (0, 1):
                @pl.when(ci % 2 == s_)
                def _(s_=s_):
                    cur, oth = s_, 1 - s_

                    @pl.when(ci + 1 < nchunk)
                    def _():
                        idx_copy(ci + 1, oth, False)
                        gathers(oth, True)

                    gathers(cur, False)
                    @pl.when(ci + 2 < nchunk)
                    def _():
                        idx_copy(ci + 2, cur, True)

                    @pl.when(ci >= 2)
                    def _():
                        out_drain(cur)

                    compute(cur)
                    out_fire(ci, cur)
            return carry

        lax.fori_loop(0, nchunk, body, 0)

        out_drain(0)
        out_drain(1)

    return k(idxa, idxb, table)


def kernel(x, tables, offsets):
    xi_t = (x + offsets[None, :]).T
    iu, ju = np.triu_indices(_F, k=1)
    iu = jnp.asarray(iu, jnp.int32)
    ju = jnp.asarray(ju, jnp.int32)
    idxa = (xi_t[iu] + (ju * _V)[:, None]).reshape(_NPAIR * _B)
    idxb = (xi_t[ju] + (iu * _V)[:, None]).reshape(_NPAIR * _B)
    table = tables.reshape(_F * _V, _D)
    out = _sc_ffm(idxa, idxb, table)
    return jnp.transpose(out, (2, 0, 1))
